# Initial kernel scaffold; baseline (speedup 1.0000x reference)
import jax
import jax.numpy as jnp
from jax.experimental import pallas as pl


def kernel(x, edge_index, edge_attr, Wl1, Wr1, We1, att1, b1, Wl2, Wr2, We2, att2, b2, Wl3, Wr3, We3, att3, b3):
    raise NotImplementedError("write your pallas kernel here")


# speedup vs baseline: 4.8446x; 4.8446x over previous
import functools

import jax
import jax.numpy as jnp
from jax import lax
from jax.experimental import pallas as pl
from jax.experimental.pallas import tpu as pltpu
from jax.experimental.pallas import tpu_sc as plsc

N = 10000
D = 128
E = 320000
NP = 10240
NC = 2
NS = 16
B = 128

E0CH = 2560
E0P = E0CH * B
E2 = E + N
E2CH = 2592
E2P = E2CH * B

_mesh = plsc.VectorSubcoreMesh(core_axis_name="c", subcore_axis_name="s")
_sc_params = pltpu.CompilerParams(needs_layout_passes=False)
_f32 = jnp.float32


def _wid():
    return lax.axis_index("c") * NS + lax.axis_index("s")


def _lane():
    return lax.iota(jnp.int32, 16)


_GDN = lax.GatherDimensionNumbers(offset_dims=(), collapsed_slice_dims=(0,),
                                  start_index_map=(0,))


def _take16(vec, idx):
    return lax.gather(vec, idx[:, None], _GDN, (1,),
                      mode=lax.GatherScatterMode.PROMISE_IN_BOUNDS)


@functools.partial(
    pl.kernel,
    out_type=jax.ShapeDtypeStruct((NC, NP, D), _f32),
    mesh=_mesh,
    compiler_params=_sc_params,
    scratch_types=[
        pltpu.VMEM((B,), jnp.int32),
        pltpu.VMEM((4 * B,), _f32),
        pltpu.VMEM((B, D), _f32),
        pltpu.VMEM_SHARED((NP, D), _f32),
    ],
)
def _p0(dst_hbm, eaf_hbm, out_hbm, dst_v, eaf_v, rows_v, acc_sh):
    cid = lax.axis_index("c")
    sid = lax.axis_index("s")
    w = _wid()
    lane = _lane()
    def _z(i, _):
        for ch in range(8):
            rows_v[i, pl.ds(16 * ch, 16)] = jnp.zeros((16,), _f32)
        return 0
    lax.fori_loop(0, B, _z, 0)
    for i in range(5):
        pltpu.sync_copy(rows_v, acc_sh.at[pl.ds(sid * 640 + i * B, B)])
    plsc.subcore_barrier()

    unit4 = jnp.where(lane == 4, 1.0, 0.0).astype(_f32)
    m03 = lane < 4
    idx_base = lane & 3

    def _chunk(t, _):
        c = w + 32 * t
        base = c * B
        pltpu.sync_copy(dst_hbm.at[pl.ds(base, B)], dst_v)
        pltpu.sync_copy(eaf_hbm.at[pl.ds(base * 4, 4 * B)], eaf_v)

        def _grp(g, _):
            vec = eaf_v[pl.ds(g * 16, 16)]
            for l in range(4):
                val = _take16(vec, idx_base + 4 * l)
                rows_v[g * 4 + l, pl.ds(0, 16)] = jnp.where(m03, val, unit4)
            return 0
        lax.fori_loop(0, B // 4, _grp, 0)
        pltpu.sync_copy(rows_v, acc_sh.at[dst_v], add=True)
        return 0
    lax.fori_loop(0, E0CH // 32, _chunk, 0)

    plsc.subcore_barrier()
    for i in range(5):
        pltpu.sync_copy(acc_sh.at[pl.ds(sid * 640 + i * B, B)], rows_v)
        pltpu.sync_copy(rows_v, out_hbm.at[cid, pl.ds(sid * 640 + i * B, B)])


@functools.partial(
    pl.kernel,
    out_type=jax.ShapeDtypeStruct((E2P, 16), _f32),
    mesh=_mesh,
    compiler_params=_sc_params,
    scratch_types=[
        pltpu.VMEM((B,), jnp.int32),
        pltpu.VMEM((B,), jnp.int32),
        pltpu.VMEM((4 * B,), _f32),
        pltpu.VMEM((B, D), _f32),
        pltpu.VMEM((B, D), _f32),
        pltpu.VMEM((B, 16), _f32),
        pltpu.VMEM((4, D), _f32),
        pltpu.VMEM((D,), _f32),
        pltpu.SemaphoreType.DMA,
        pltpu.SemaphoreType.DMA,
    ],
)
def _p1(src_hbm, dst_hbm, eaf_hbm, xl_hbm, xr_hbm, we_hbm, att_hbm, a16_hbm,
        src_v, dst_v, eaf_v, xj_v, xi_v, a16_v, we_v, att_v, sem1, sem2):
    w = _wid()
    pltpu.sync_copy(we_hbm, we_v)
    pltpu.sync_copy(att_hbm, att_v)
    wec = [[we_v[k, pl.ds(16 * ch, 16)] for ch in range(8)] for k in range(4)]
    attc = [att_v[pl.ds(16 * ch, 16)] for ch in range(8)]

    def _chunk(t, _):
        c = w + 32 * t
        base = c * B
        pltpu.sync_copy(src_hbm.at[pl.ds(base, B)], src_v)
        pltpu.sync_copy(dst_hbm.at[pl.ds(base, B)], dst_v)
        pltpu.sync_copy(eaf_hbm.at[pl.ds(base * 4, 4 * B)], eaf_v)
        d1 = pltpu.async_copy(xl_hbm.at[src_v], xj_v, sem1)
        d2 = pltpu.async_copy(xr_hbm.at[dst_v], xi_v, sem2)
        d1.wait()
        d2.wait()

        def _grp(g, _):
            vecs = [eaf_v[pl.ds(g * 64 + 16 * m, 16)] for m in range(4)]
            for l in range(16):
                j = g * 16 + l
                vec = vecs[l // 4]
                eb = [_take16(vec, jnp.full((16,), 4 * (l % 4) + k, jnp.int32))
                      for k in range(4)]
                acc = jnp.zeros((16,), _f32)
                for ch in range(8):
                    u = xj_v[j, pl.ds(16 * ch, 16)] + xi_v[j, pl.ds(16 * ch, 16)]
                    u = u + (eb[0] * wec[0][ch] + eb[1] * wec[1][ch]
                             + eb[2] * wec[2][ch] + eb[3] * wec[3][ch])
                    u = jnp.maximum(u, 0.2 * u)
                    acc = acc + u * attc[ch]
                a16_v[j] = acc
            return 0
        lax.fori_loop(0, B // 16, _grp, 0)
        pltpu.sync_copy(a16_v, a16_hbm.at[pl.ds(base, B)])
        return 0
    lax.fori_loop(0, E2CH // 32, _chunk, 0)


@functools.partial(
    pl.kernel,
    out_type=[jax.ShapeDtypeStruct((NC, NP, D), _f32),
              jax.ShapeDtypeStruct((NC, NP, D), _f32)],
    mesh=_mesh,
    compiler_params=_sc_params,
    scratch_types=[
        pltpu.VMEM((B,), jnp.int32),
        pltpu.VMEM((B,), jnp.int32),
        pltpu.VMEM((B,), _f32),
        pltpu.VMEM((B, D), _f32),
        pltpu.VMEM((B, D), _f32),
        pltpu.VMEM_SHARED((NP, D), _f32),
        pltpu.SemaphoreType.DMA,
    ],
)
def _p2(src_hbm, dst_hbm, ax_hbm, xl_hbm, out_hbm, asum_hbm,
        src_v, dst_v, ax_v, xj_v, r_v, acc_sh, sem1):
    cid = lax.axis_index("c")
    sid = lax.axis_index("s")
    w = _wid()
    lane = _lane()

    def _z(i, _):
        for ch in range(8):
            r_v[i, pl.ds(16 * ch, 16)] = jnp.zeros((16,), _f32)
        return 0
    lax.fori_loop(0, B, _z, 0)

    for i in range(5):
        pltpu.sync_copy(r_v, acc_sh.at[pl.ds(sid * 640 + i * B, B)])
    plsc.subcore_barrier()

    def _chunk(t, _):
        c = w + 32 * t
        base = c * B
        pltpu.sync_copy(src_hbm.at[pl.ds(base, B)], src_v)
        pltpu.sync_copy(dst_hbm.at[pl.ds(base, B)], dst_v)
        pltpu.sync_copy(ax_hbm.at[pl.ds(base, B)], ax_v)
        pltpu.async_copy(xl_hbm.at[src_v], xj_v, sem1).wait()

        def _grp(g, _):
            axg = ax_v[pl.ds(g * 16, 16)]
            for l in range(16):
                j = g * 16 + l
                axb = _take16(axg, jnp.full((16,), l, jnp.int32))
                for ch in range(8):
                    r_v[j, pl.ds(16 * ch, 16)] = xj_v[j, pl.ds(16 * ch, 16)] * axb
            return 0
        lax.fori_loop(0, B // 16, _grp, 0)
        pltpu.sync_copy(r_v, acc_sh.at[dst_v], add=True)
        return 0
    lax.fori_loop(0, E2CH // 32, _chunk, 0)

    plsc.subcore_barrier()
    for i in range(5):
        pltpu.sync_copy(acc_sh.at[pl.ds(sid * 640 + i * B, B)], r_v)
        pltpu.sync_copy(r_v, out_hbm.at[cid, pl.ds(sid * 640 + i * B, B)])
    plsc.subcore_barrier()

    def _zb(i, _):
        for ch in range(8):
            r_v[i, pl.ds(16 * ch, 16)] = jnp.zeros((16,), _f32)
        return 0
    lax.fori_loop(0, B, _zb, 0)
    for i in range(5):
        pltpu.sync_copy(r_v, acc_sh.at[pl.ds(sid * 640 + i * B, B)])
    plsc.subcore_barrier()

    def _chunk_b(t, _):
        c = w + 32 * t
        base = c * B
        pltpu.sync_copy(dst_hbm.at[pl.ds(base, B)], dst_v)
        pltpu.sync_copy(ax_hbm.at[pl.ds(base, B)], ax_v)

        def _grp(g, _):
            axg = ax_v[pl.ds(g * 16, 16)]
            for l in range(16):
                axb = _take16(axg, jnp.full((16,), l, jnp.int32))
                r_v[g * 16 + l, pl.ds(0, 16)] = jnp.where(lane == 0, axb, 0.0)
            return 0
        lax.fori_loop(0, B // 16, _grp, 0)
        pltpu.sync_copy(r_v, acc_sh.at[dst_v], add=True)
        return 0
    lax.fori_loop(0, E2CH // 32, _chunk_b, 0)

    plsc.subcore_barrier()
    for i in range(5):
        pltpu.sync_copy(acc_sh.at[pl.ds(sid * 640 + i * B, B)], r_v)
        pltpu.sync_copy(r_v, asum_hbm.at[cid, pl.ds(sid * 640 + i * B, B)])


def _t0_body(h_ref, wl_ref, wr_ref, xl_ref, xr_ref):
    h = h_ref[...]
    dn = (((1,), (0,)), ((), ()))
    xl_ref[...] = lax.dot_general(h, wl_ref[...], dn,
                                  precision=lax.Precision.HIGHEST,
                                  preferred_element_type=_f32)
    xr_ref[...] = lax.dot_general(h, wr_ref[...], dn,
                                  precision=lax.Precision.HIGHEST,
                                  preferred_element_type=_f32)


def _t0(h, wl, wr):
    return pl.pallas_call(
        _t0_body,
        grid=(NP // 256,),
        in_specs=[pl.BlockSpec((256, D), lambda i: (i, 0)),
                  pl.BlockSpec((D, D), lambda i: (0, 0)),
                  pl.BlockSpec((D, D), lambda i: (0, 0))],
        out_specs=[pl.BlockSpec((256, D), lambda i: (i, 0)),
                   pl.BlockSpec((256, D), lambda i: (i, 0))],
        out_shape=[jax.ShapeDtypeStruct((NP, D), _f32),
                   jax.ShapeDtypeStruct((NP, D), _f32)],
    )(h, wl, wr)


def _t1_body(a16_ref, al_ref, gmax_ref, scr):
    i = pl.program_id(0)
    blk = a16_ref[...]
    s = jnp.sum(blk, axis=1)
    al_ref[...] = s

    @pl.when(i == 0)
    def _():
        scr[0, 0] = -3.0e38
    scr[0, 0] = jnp.maximum(scr[0, 0], jnp.max(s))

    @pl.when(i == pl.num_programs(0) - 1)
    def _():
        gmax_ref[0, 0] = scr[0, 0]


def _t1(a16):
    return pl.pallas_call(
        _t1_body,
        grid=(E2P // 4096,),
        in_specs=[pl.BlockSpec((4096, 16), lambda i: (i, 0))],
        out_specs=[pl.BlockSpec((4096,), lambda i: (i,)),
                   pl.BlockSpec(memory_space=pltpu.SMEM)],
        out_shape=[jax.ShapeDtypeStruct((E2P,), _f32),
                   jax.ShapeDtypeStruct((1, 1), _f32)],
        scratch_shapes=[pltpu.SMEM((1, 1), _f32)],
    )(a16)


def _t2_body(al_ref, g_ref, ax_ref):
    ax_ref[...] = jnp.exp(al_ref[...] - g_ref[0, 0])


def _t2(alpha, gmax):
    return pl.pallas_call(
        _t2_body,
        grid=(E2P // 4096,),
        in_specs=[pl.BlockSpec((4096,), lambda i: (i,)),
                  pl.BlockSpec(memory_space=pltpu.SMEM)],
        out_specs=pl.BlockSpec((4096,), lambda i: (i,)),
        out_shape=jax.ShapeDtypeStruct((E2P,), _f32),
    )(alpha, gmax)


def _tla_body(p0_ref, p1_ref, la_ref):
    s = p0_ref[:, :16] + p1_ref[:, :16]
    cnt = jnp.maximum(s[:, 4:5], 1.0)
    col = lax.broadcasted_iota(jnp.int32, s.shape, 1)
    la_ref[...] = jnp.where(col < 4, s / cnt, 0.0)


def _tla(p0, p1):
    return pl.pallas_call(
        _tla_body,
        grid=(NP // 512,),
        in_specs=[pl.BlockSpec((512, D), lambda i: (i, 0)),
                  pl.BlockSpec((512, D), lambda i: (i, 0))],
        out_specs=pl.BlockSpec((512, 16), lambda i: (i, 0)),
        out_shape=jax.ShapeDtypeStruct((NP, 16), _f32),
    )(p0, p1)


def _t3_body(relu, pa_ref, pb_ref, qa_ref, qb_ref, b_ref, h_ref):
    acc = pa_ref[...] + pb_ref[...]
    den = qa_ref[:, 0:1] + qb_ref[:, 0:1] + 1e-16
    h = acc / den + b_ref[...]
    if relu:
        h = jnp.maximum(h, 0.0)
    h_ref[...] = h


def _t3(pa, pb, qa, qb, bias, relu):
    return pl.pallas_call(
        functools.partial(_t3_body, relu),
        grid=(NP // 512,),
        in_specs=[pl.BlockSpec((512, D), lambda i: (i, 0)),
                  pl.BlockSpec((512, D), lambda i: (i, 0)),
                  pl.BlockSpec((512, D), lambda i: (i, 0)),
                  pl.BlockSpec((512, D), lambda i: (i, 0)),
                  pl.BlockSpec((1, D), lambda i: (0, 0))],
        out_specs=pl.BlockSpec((512, D), lambda i: (i, 0)),
        out_shape=jax.ShapeDtypeStruct((NP, D), _f32),
    )(pa, pb, qa, qb, bias)


def _layer(h, src2, dst2, eaf2, wl, wr, we, att, bias, relu):
    xl, xr = _t0(h, wl, wr)
    a16 = _p1(src2, dst2, eaf2, xl, xr, we, att)
    alpha, gmax = _t1(a16)
    aexp = _t2(alpha, gmax)
    part, asw = _p2(src2, dst2, aexp, xl)
    return _t3(part[0], part[1], asw[0], asw[1], bias, relu)


def kernel(x, edge_index, edge_attr,
           Wl1, Wr1, We1, att1, b1,
           Wl2, Wr2, We2, att2, b2,
           Wl3, Wr3, We3, att3, b3):
    src = edge_index[0].astype(jnp.int32)
    dst = edge_index[1].astype(jnp.int32)
    ea = edge_attr.astype(_f32)

    dst0 = jnp.concatenate([dst, jnp.full((E0P - E,), N, jnp.int32)])
    eaf0 = jnp.concatenate([ea.reshape(-1), jnp.zeros(((E0P - E) * 4,), _f32)])
    part0 = _p0(dst0, eaf0)
    la16 = _tla(part0[0], part0[1])
    la = la16[:N, :4]

    loop_idx = jnp.arange(N, dtype=jnp.int32)
    padi = jnp.full((E2P - E2,), N, jnp.int32)
    src2 = jnp.concatenate([src, loop_idx, padi])
    dst2 = jnp.concatenate([dst, loop_idx, padi])
    eaf2 = jnp.concatenate([ea.reshape(-1), la.reshape(-1),
                            jnp.zeros(((E2P - E2) * 4,), _f32)])

    h = jnp.pad(x.astype(_f32), ((0, NP - N), (0, 0)))
    layers = [(Wl1, Wr1, We1, att1, b1, True),
              (Wl2, Wr2, We2, att2, b2, True),
              (Wl3, Wr3, We3, att3, b3, False)]
    for wl, wr, we, att, bias, relu in layers:
        h = _layer(h, src2, dst2, eaf2, wl.astype(_f32), wr.astype(_f32),
                   we.astype(_f32), att.reshape(-1).astype(_f32),
                   bias.reshape(1, D).astype(_f32), relu)
    return h[:N]


# speedup vs baseline: 5.5823x; 1.1523x over previous
import functools

import jax
import jax.numpy as jnp
from jax import lax
from jax.experimental import pallas as pl
from jax.experimental.pallas import tpu as pltpu
from jax.experimental.pallas import tpu_sc as plsc

N = 10000
D = 128
E = 320000
NP = 10240
NC = 2
NS = 16
B = 128

E0CH = 2560
E0P = E0CH * B
E2 = E + N
E2CH = 2592
E2P = E2CH * B

_mesh = plsc.VectorSubcoreMesh(core_axis_name="c", subcore_axis_name="s")
_sc_params = pltpu.CompilerParams(needs_layout_passes=False)
_f32 = jnp.float32


def _wid():
    return lax.axis_index("c") * NS + lax.axis_index("s")


def _lane():
    return lax.iota(jnp.int32, 16)


_GDN = lax.GatherDimensionNumbers(offset_dims=(), collapsed_slice_dims=(0,),
                                  start_index_map=(0,))


def _take16(vec, idx):
    return lax.gather(vec, idx[:, None], _GDN, (1,),
                      mode=lax.GatherScatterMode.PROMISE_IN_BOUNDS)


@functools.partial(
    pl.kernel,
    out_type=jax.ShapeDtypeStruct((NC, NP, D), _f32),
    mesh=_mesh,
    compiler_params=_sc_params,
    scratch_types=[
        pltpu.VMEM((B,), jnp.int32),
        pltpu.VMEM((4 * B,), _f32),
        pltpu.VMEM((B, D), _f32),
        pltpu.VMEM_SHARED((NP, D), _f32),
    ],
)
def _p0(dst_hbm, eaf_hbm, out_hbm, dst_v, eaf_v, rows_v, acc_sh):
    cid = lax.axis_index("c")
    sid = lax.axis_index("s")
    w = _wid()
    lane = _lane()
    def _z(i, _):
        for ch in range(8):
            rows_v[i, pl.ds(16 * ch, 16)] = jnp.zeros((16,), _f32)
        return 0
    lax.fori_loop(0, B, _z, 0)
    for i in range(5):
        pltpu.sync_copy(rows_v, acc_sh.at[pl.ds(sid * 640 + i * B, B)])
    plsc.subcore_barrier()

    unit4 = jnp.where(lane == 4, 1.0, 0.0).astype(_f32)
    m03 = lane < 4
    idx_base = lane & 3

    def _chunk(t, _):
        c = w + 32 * t
        base = c * B
        pltpu.sync_copy(dst_hbm.at[pl.ds(base, B)], dst_v)
        pltpu.sync_copy(eaf_hbm.at[pl.ds(base * 4, 4 * B)], eaf_v)

        def _grp(g, _):
            vec = eaf_v[pl.ds(g * 16, 16)]
            for l in range(4):
                val = _take16(vec, idx_base + 4 * l)
                rows_v[g * 4 + l, pl.ds(0, 16)] = jnp.where(m03, val, unit4)
            return 0
        lax.fori_loop(0, B // 4, _grp, 0)
        pltpu.sync_copy(rows_v, acc_sh.at[dst_v], add=True)
        return 0
    lax.fori_loop(0, E0CH // 32, _chunk, 0)

    plsc.subcore_barrier()
    for i in range(5):
        pltpu.sync_copy(acc_sh.at[pl.ds(sid * 640 + i * B, B)], rows_v)
        pltpu.sync_copy(rows_v, out_hbm.at[cid, pl.ds(sid * 640 + i * B, B)])


@functools.partial(
    pl.kernel,
    out_type=jax.ShapeDtypeStruct((E2P, 16), _f32),
    mesh=_mesh,
    compiler_params=_sc_params,
    scratch_types=[
        pltpu.VMEM((B,), jnp.int32),
        pltpu.VMEM((B,), jnp.int32),
        pltpu.VMEM((B,), jnp.int32),
        pltpu.VMEM((B,), jnp.int32),
        pltpu.VMEM((4 * B,), _f32),
        pltpu.VMEM((4 * B,), _f32),
        pltpu.VMEM((B, D), _f32),
        pltpu.VMEM((B, D), _f32),
        pltpu.VMEM((B, D), _f32),
        pltpu.VMEM((B, D), _f32),
        pltpu.VMEM((B, 16), _f32),
        pltpu.VMEM((4, D), _f32),
        pltpu.VMEM((D,), _f32),
        pltpu.SemaphoreType.DMA,
        pltpu.SemaphoreType.DMA,
        pltpu.SemaphoreType.DMA,
        pltpu.SemaphoreType.DMA,
    ],
)
def _p1(src_hbm, dst_hbm, eaf_hbm, xl_hbm, xr_hbm, we_hbm, att_hbm, a16_hbm,
        src0, src1, dst0, dst1, eaf0, eaf1, xj0, xj1, xi0, xi1,
        a16_v, we_v, att_v, sj0, sj1, si0, si1):
    w = _wid()
    pltpu.sync_copy(we_hbm, we_v)
    pltpu.sync_copy(att_hbm, att_v)
    wec = [[we_v[k, pl.ds(16 * ch, 16)] for ch in range(8)] for k in range(4)]
    attc = [att_v[pl.ds(16 * ch, 16)] for ch in range(8)]
    bufs = ((src0, dst0, eaf0, xj0, xi0, sj0, si0),
            (src1, dst1, eaf1, xj1, xi1, sj1, si1))

    def _prefetch(t, buf):
        srcv, dstv, eafv, xjv, xiv, sj, si = buf
        base = (w + 32 * t) * B
        pltpu.sync_copy(src_hbm.at[pl.ds(base, B)], srcv)
        pltpu.sync_copy(dst_hbm.at[pl.ds(base, B)], dstv)
        pltpu.sync_copy(eaf_hbm.at[pl.ds(base * 4, 4 * B)], eafv)
        pltpu.async_copy(xl_hbm.at[srcv], xjv, sj)
        pltpu.async_copy(xr_hbm.at[dstv], xiv, si)

    def _compute(t, buf):
        srcv, dstv, eafv, xjv, xiv, sj, si = buf
        base = (w + 32 * t) * B
        pltpu.make_async_copy(xl_hbm.at[srcv], xjv, sj).wait()
        pltpu.make_async_copy(xr_hbm.at[dstv], xiv, si).wait()

        def _grp(g, _):
            vecs = [eafv[pl.ds(g * 64 + 16 * m, 16)] for m in range(4)]
            for l in range(16):
                j = g * 16 + l
                vec = vecs[l // 4]
                eb = [_take16(vec, jnp.full((16,), 4 * (l % 4) + k, jnp.int32))
                      for k in range(4)]
                acc = jnp.zeros((16,), _f32)
                for ch in range(8):
                    u = xjv[j, pl.ds(16 * ch, 16)] + xiv[j, pl.ds(16 * ch, 16)]
                    u = u + (eb[0] * wec[0][ch] + eb[1] * wec[1][ch]
                             + eb[2] * wec[2][ch] + eb[3] * wec[3][ch])
                    u = jnp.maximum(u, 0.2 * u)
                    acc = acc + u * attc[ch]
                a16_v[j] = acc
            return 0
        lax.fori_loop(0, B // 16, _grp, 0)
        pltpu.sync_copy(a16_v, a16_hbm.at[pl.ds(base, B)])

    _prefetch(0, bufs[0])

    def _pair(p, _):
        _prefetch(2 * p + 1, bufs[1])
        _compute(2 * p, bufs[0])
        _prefetch(2 * p + 2, bufs[0])
        _compute(2 * p + 1, bufs[1])
        return 0
    lax.fori_loop(0, (E2CH // 32 - 1) // 2, _pair, 0)
    _compute(E2CH // 32 - 1, bufs[0])


@functools.partial(
    pl.kernel,
    out_type=[jax.ShapeDtypeStruct((NC, NP, D), _f32),
              jax.ShapeDtypeStruct((NC, NP, D), _f32)],
    mesh=_mesh,
    compiler_params=_sc_params,
    scratch_types=[
        pltpu.VMEM((B,), jnp.int32),
        pltpu.VMEM((B,), jnp.int32),
        pltpu.VMEM((B,), jnp.int32),
        pltpu.VMEM((B,), jnp.int32),
        pltpu.VMEM((B,), _f32),
        pltpu.VMEM((B,), _f32),
        pltpu.VMEM((B, D), _f32),
        pltpu.VMEM((B, D), _f32),
        pltpu.VMEM_SHARED((NP, D), _f32),
        pltpu.SemaphoreType.DMA,
        pltpu.SemaphoreType.DMA,
    ],
)
def _p2(src_hbm, dst_hbm, ax_hbm, xl_hbm, out_hbm, asum_hbm,
        src0, src1, dst0, dst1, ax0, ax1, xj0, xj1, acc_sh, sj0, sj1):
    cid = lax.axis_index("c")
    sid = lax.axis_index("s")
    w = _wid()
    lane = _lane()

    def _zr(i, _):
        for ch in range(8):
            xj0[i, pl.ds(16 * ch, 16)] = jnp.zeros((16,), _f32)
        return 0
    lax.fori_loop(0, B, _zr, 0)
    for i in range(5):
        pltpu.sync_copy(xj0, acc_sh.at[pl.ds(sid * 640 + i * B, B)])
    plsc.subcore_barrier()

    bufs = ((src0, dst0, ax0, xj0, sj0), (src1, dst1, ax1, xj1, sj1))

    def _prefetch(t, buf):
        srcv, dstv, axv, xjv, sj = buf
        base = (w + 32 * t) * B
        pltpu.sync_copy(src_hbm.at[pl.ds(base, B)], srcv)
        pltpu.sync_copy(dst_hbm.at[pl.ds(base, B)], dstv)
        pltpu.sync_copy(ax_hbm.at[pl.ds(base, B)], axv)
        pltpu.async_copy(xl_hbm.at[srcv], xjv, sj)

    def _compute(t, buf):
        srcv, dstv, axv, xjv, sj = buf
        pltpu.make_async_copy(xl_hbm.at[srcv], xjv, sj).wait()

        def _grp(g, _):
            axg = axv[pl.ds(g * 16, 16)]
            for l in range(16):
                j = g * 16 + l
                axb = _take16(axg, jnp.full((16,), l, jnp.int32))
                for ch in range(8):
                    xjv[j, pl.ds(16 * ch, 16)] = xjv[j, pl.ds(16 * ch, 16)] * axb
            return 0
        lax.fori_loop(0, B // 16, _grp, 0)
        pltpu.sync_copy(xjv, acc_sh.at[dstv], add=True)

    _prefetch(0, bufs[0])

    def _pair(p, _):
        _prefetch(2 * p + 1, bufs[1])
        _compute(2 * p, bufs[0])
        _prefetch(2 * p + 2, bufs[0])
        _compute(2 * p + 1, bufs[1])
        return 0
    lax.fori_loop(0, (E2CH // 32 - 1) // 2, _pair, 0)
    _compute(E2CH // 32 - 1, bufs[0])

    plsc.subcore_barrier()
    for i in range(5):
        pltpu.sync_copy(acc_sh.at[pl.ds(sid * 640 + i * B, B)], xj0)
        pltpu.sync_copy(xj0, out_hbm.at[cid, pl.ds(sid * 640 + i * B, B)])
    plsc.subcore_barrier()

    def _zb(i, _):
        for ch in range(8):
            xj0[i, pl.ds(16 * ch, 16)] = jnp.zeros((16,), _f32)
        return 0
    lax.fori_loop(0, B, _zb, 0)
    for i in range(5):
        pltpu.sync_copy(xj0, acc_sh.at[pl.ds(sid * 640 + i * B, B)])
    plsc.subcore_barrier()

    def _chunk_b(t, _):
        base = (w + 32 * t) * B
        pltpu.sync_copy(dst_hbm.at[pl.ds(base, B)], dst0)
        pltpu.sync_copy(ax_hbm.at[pl.ds(base, B)], ax0)

        def _grp(g, _):
            axg = ax0[pl.ds(g * 16, 16)]
            for l in range(16):
                axb = _take16(axg, jnp.full((16,), l, jnp.int32))
                xj0[g * 16 + l, pl.ds(0, 16)] = jnp.where(lane == 0, axb, 0.0)
            return 0
        lax.fori_loop(0, B // 16, _grp, 0)
        pltpu.sync_copy(xj0, acc_sh.at[dst0], add=True)
        return 0
    lax.fori_loop(0, E2CH // 32, _chunk_b, 0)

    plsc.subcore_barrier()
    for i in range(5):
        pltpu.sync_copy(acc_sh.at[pl.ds(sid * 640 + i * B, B)], xj0)
        pltpu.sync_copy(xj0, asum_hbm.at[cid, pl.ds(sid * 640 + i * B, B)])


def _t0_body(h_ref, wl_ref, wr_ref, xl_ref, xr_ref):
    h = h_ref[...]
    dn = (((1,), (0,)), ((), ()))
    xl_ref[...] = lax.dot_general(h, wl_ref[...], dn,
                                  precision=lax.Precision.HIGHEST,
                                  preferred_element_type=_f32)
    xr_ref[...] = lax.dot_general(h, wr_ref[...], dn,
                                  precision=lax.Precision.HIGHEST,
                                  preferred_element_type=_f32)


def _t0(h, wl, wr):
    return pl.pallas_call(
        _t0_body,
        grid=(NP // 256,),
        in_specs=[pl.BlockSpec((256, D), lambda i: (i, 0)),
                  pl.BlockSpec((D, D), lambda i: (0, 0)),
                  pl.BlockSpec((D, D), lambda i: (0, 0))],
        out_specs=[pl.BlockSpec((256, D), lambda i: (i, 0)),
                   pl.BlockSpec((256, D), lambda i: (i, 0))],
        out_shape=[jax.ShapeDtypeStruct((NP, D), _f32),
                   jax.ShapeDtypeStruct((NP, D), _f32)],
    )(h, wl, wr)


def _t1_body(a16_ref, al_ref, gmax_ref, scr):
    i = pl.program_id(0)
    blk = a16_ref[...]
    s = jnp.sum(blk, axis=1)
    al_ref[...] = s

    @pl.when(i == 0)
    def _():
        scr[0, 0] = -3.0e38
    scr[0, 0] = jnp.maximum(scr[0, 0], jnp.max(s))

    @pl.when(i == pl.num_programs(0) - 1)
    def _():
        gmax_ref[0, 0] = scr[0, 0]


def _t1(a16):
    return pl.pallas_call(
        _t1_body,
        grid=(E2P // 4096,),
        in_specs=[pl.BlockSpec((4096, 16), lambda i: (i, 0))],
        out_specs=[pl.BlockSpec((4096,), lambda i: (i,)),
                   pl.BlockSpec(memory_space=pltpu.SMEM)],
        out_shape=[jax.ShapeDtypeStruct((E2P,), _f32),
                   jax.ShapeDtypeStruct((1, 1), _f32)],
        scratch_shapes=[pltpu.SMEM((1, 1), _f32)],
    )(a16)


def _t2_body(al_ref, g_ref, ax_ref):
    ax_ref[...] = jnp.exp(al_ref[...] - g_ref[0, 0])


def _t2(alpha, gmax):
    return pl.pallas_call(
        _t2_body,
        grid=(E2P // 4096,),
        in_specs=[pl.BlockSpec((4096,), lambda i: (i,)),
                  pl.BlockSpec(memory_space=pltpu.SMEM)],
        out_specs=pl.BlockSpec((4096,), lambda i: (i,)),
        out_shape=jax.ShapeDtypeStruct((E2P,), _f32),
    )(alpha, gmax)


def _tla_body(p0_ref, p1_ref, la_ref):
    s = p0_ref[:, :16] + p1_ref[:, :16]
    cnt = jnp.maximum(s[:, 4:5], 1.0)
    col = lax.broadcasted_iota(jnp.int32, s.shape, 1)
    la_ref[...] = jnp.where(col < 4, s / cnt, 0.0)


def _tla(p0, p1):
    return pl.pallas_call(
        _tla_body,
        grid=(NP // 512,),
        in_specs=[pl.BlockSpec((512, D), lambda i: (i, 0)),
                  pl.BlockSpec((512, D), lambda i: (i, 0))],
        out_specs=pl.BlockSpec((512, 16), lambda i: (i, 0)),
        out_shape=jax.ShapeDtypeStruct((NP, 16), _f32),
    )(p0, p1)


def _t3_body(relu, pa_ref, pb_ref, qa_ref, qb_ref, b_ref, h_ref):
    acc = pa_ref[...] + pb_ref[...]
    den = qa_ref[:, 0:1] + qb_ref[:, 0:1] + 1e-16
    h = acc / den + b_ref[...]
    if relu:
        h = jnp.maximum(h, 0.0)
    h_ref[...] = h


def _t3(pa, pb, qa, qb, bias, relu):
    return pl.pallas_call(
        functools.partial(_t3_body, relu),
        grid=(NP // 512,),
        in_specs=[pl.BlockSpec((512, D), lambda i: (i, 0)),
                  pl.BlockSpec((512, D), lambda i: (i, 0)),
                  pl.BlockSpec((512, D), lambda i: (i, 0)),
                  pl.BlockSpec((512, D), lambda i: (i, 0)),
                  pl.BlockSpec((1, D), lambda i: (0, 0))],
        out_specs=pl.BlockSpec((512, D), lambda i: (i, 0)),
        out_shape=jax.ShapeDtypeStruct((NP, D), _f32),
    )(pa, pb, qa, qb, bias)


def _layer(h, src2, dst2, eaf2, wl, wr, we, att, bias, relu):
    xl, xr = _t0(h, wl, wr)
    a16 = _p1(src2, dst2, eaf2, xl, xr, we, att)
    alpha, gmax = _t1(a16)
    aexp = _t2(alpha, gmax)
    part, asw = _p2(src2, dst2, aexp, xl)
    return _t3(part[0], part[1], asw[0], asw[1], bias, relu)


def kernel(x, edge_index, edge_attr,
           Wl1, Wr1, We1, att1, b1,
           Wl2, Wr2, We2, att2, b2,
           Wl3, Wr3, We3, att3, b3):
    src = edge_index[0].astype(jnp.int32)
    dst = edge_index[1].astype(jnp.int32)
    ea = edge_attr.astype(_f32)

    dst0 = jnp.concatenate([dst, jnp.full((E0P - E,), N, jnp.int32)])
    eaf0 = jnp.concatenate([ea.reshape(-1), jnp.zeros(((E0P - E) * 4,), _f32)])
    part0 = _p0(dst0, eaf0)
    la16 = _tla(part0[0], part0[1])
    la = la16[:N, :4]

    loop_idx = jnp.arange(N, dtype=jnp.int32)
    padi = jnp.full((E2P - E2,), N, jnp.int32)
    src2 = jnp.concatenate([src, loop_idx, padi])
    dst2 = jnp.concatenate([dst, loop_idx, padi])
    eaf2 = jnp.concatenate([ea.reshape(-1), la.reshape(-1),
                            jnp.zeros(((E2P - E2) * 4,), _f32)])

    h = jnp.pad(x.astype(_f32), ((0, NP - N), (0, 0)))
    layers = [(Wl1, Wr1, We1, att1, b1, True),
              (Wl2, Wr2, We2, att2, b2, True),
              (Wl3, Wr3, We3, att3, b3, False)]
    for wl, wr, we, att, bias, relu in layers:
        h = _layer(h, src2, dst2, eaf2, wl.astype(_f32), wr.astype(_f32),
                   we.astype(_f32), att.reshape(-1).astype(_f32),
                   bias.reshape(1, D).astype(_f32), relu)
    return h[:N]


# speedup vs baseline: 7.0943x; 1.2709x over previous
import functools

import jax
import jax.numpy as jnp
from jax import lax
from jax.experimental import pallas as pl
from jax.experimental.pallas import tpu as pltpu
from jax.experimental.pallas import tpu_sc as plsc

N = 10000
D = 128
E = 320000
NP = 10240
NC = 2
NS = 16
B = 128

E0CH = 2560
E0P = E0CH * B
E2 = E + N
E2CH = 2592
E2P = E2CH * B

_mesh = plsc.VectorSubcoreMesh(core_axis_name="c", subcore_axis_name="s")
_sc_params = pltpu.CompilerParams(needs_layout_passes=False)
_f32 = jnp.float32


def _wid():
    return lax.axis_index("c") * NS + lax.axis_index("s")


def _lane():
    return lax.iota(jnp.int32, 16)


_GDN = lax.GatherDimensionNumbers(offset_dims=(), collapsed_slice_dims=(0,),
                                  start_index_map=(0,))


def _take16(vec, idx):
    return lax.gather(vec, idx[:, None], _GDN, (1,),
                      mode=lax.GatherScatterMode.PROMISE_IN_BOUNDS)


@functools.partial(
    pl.kernel,
    out_type=jax.ShapeDtypeStruct((NC, NP, D), _f32),
    mesh=_mesh,
    compiler_params=_sc_params,
    scratch_types=[
        pltpu.VMEM((B,), jnp.int32),
        pltpu.VMEM((4 * B,), _f32),
        pltpu.VMEM((B, D), _f32),
        pltpu.VMEM_SHARED((NP, D), _f32),
    ],
)
def _p0(dst_hbm, eaf_hbm, out_hbm, dst_v, eaf_v, rows_v, acc_sh):
    cid = lax.axis_index("c")
    sid = lax.axis_index("s")
    w = _wid()
    lane = _lane()
    def _z(i, _):
        for ch in range(8):
            rows_v[i, pl.ds(16 * ch, 16)] = jnp.zeros((16,), _f32)
        return 0
    lax.fori_loop(0, B, _z, 0)
    for i in range(5):
        pltpu.sync_copy(rows_v, acc_sh.at[pl.ds(sid * 640 + i * B, B)])
    plsc.subcore_barrier()

    unit4 = jnp.where(lane == 4, 1.0, 0.0).astype(_f32)
    m03 = lane < 4
    idx_base = lane & 3

    def _chunk(t, _):
        c = w + 32 * t
        base = c * B
        pltpu.sync_copy(dst_hbm.at[pl.ds(base, B)], dst_v)
        pltpu.sync_copy(eaf_hbm.at[pl.ds(base * 4, 4 * B)], eaf_v)

        def _grp(g, _):
            vec = eaf_v[pl.ds(g * 16, 16)]
            for l in range(4):
                val = _take16(vec, idx_base + 4 * l)
                rows_v[g * 4 + l, pl.ds(0, 16)] = jnp.where(m03, val, unit4)
            return 0
        lax.fori_loop(0, B // 4, _grp, 0)
        pltpu.sync_copy(rows_v, acc_sh.at[dst_v], add=True)
        return 0
    lax.fori_loop(0, E0CH // 32, _chunk, 0)

    plsc.subcore_barrier()
    for i in range(5):
        pltpu.sync_copy(acc_sh.at[pl.ds(sid * 640 + i * B, B)], rows_v)
        pltpu.sync_copy(rows_v, out_hbm.at[cid, pl.ds(sid * 640 + i * B, B)])


@functools.partial(
    pl.kernel,
    out_type=jax.ShapeDtypeStruct((E2P, 16), _f32),
    mesh=_mesh,
    compiler_params=_sc_params,
    scratch_types=[
        pltpu.VMEM((B,), jnp.int32),
        pltpu.VMEM((B,), jnp.int32),
        pltpu.VMEM((B,), jnp.int32),
        pltpu.VMEM((B,), jnp.int32),
        pltpu.VMEM((4 * B,), _f32),
        pltpu.VMEM((4 * B,), _f32),
        pltpu.VMEM((B, D), _f32),
        pltpu.VMEM((B, D), _f32),
        pltpu.VMEM((B, D), _f32),
        pltpu.VMEM((B, D), _f32),
        pltpu.VMEM((B, 16), _f32),
        pltpu.VMEM((4, D), _f32),
        pltpu.VMEM((D,), _f32),
        pltpu.SemaphoreType.DMA,
        pltpu.SemaphoreType.DMA,
        pltpu.SemaphoreType.DMA,
        pltpu.SemaphoreType.DMA,
    ],
)
def _p1(src_hbm, dst_hbm, eaf_hbm, xl_hbm, xr_hbm, we_hbm, att_hbm, a16_hbm,
        src0, src1, dst0, dst1, eaf0, eaf1, xj0, xj1, xi0, xi1,
        a16_v, we_v, att_v, sj0, sj1, si0, si1):
    w = _wid()
    pltpu.sync_copy(we_hbm, we_v)
    pltpu.sync_copy(att_hbm, att_v)
    wec = [[we_v[k, pl.ds(16 * ch, 16)] for ch in range(8)] for k in range(4)]
    attc = [att_v[pl.ds(16 * ch, 16)] for ch in range(8)]
    bufs = ((src0, dst0, eaf0, xj0, xi0, sj0, si0),
            (src1, dst1, eaf1, xj1, xi1, sj1, si1))

    def _prefetch(t, buf):
        srcv, dstv, eafv, xjv, xiv, sj, si = buf
        base = (w + 32 * t) * B
        pltpu.sync_copy(src_hbm.at[pl.ds(base, B)], srcv)
        pltpu.sync_copy(dst_hbm.at[pl.ds(base, B)], dstv)
        pltpu.sync_copy(eaf_hbm.at[pl.ds(base * 4, 4 * B)], eafv)
        pltpu.async_copy(xl_hbm.at[srcv], xjv, sj)
        pltpu.async_copy(xr_hbm.at[dstv], xiv, si)

    def _compute(t, buf):
        srcv, dstv, eafv, xjv, xiv, sj, si = buf
        base = (w + 32 * t) * B
        pltpu.make_async_copy(xl_hbm.at[srcv], xjv, sj).wait()
        pltpu.make_async_copy(xr_hbm.at[dstv], xiv, si).wait()

        @functools.partial(plsc.parallel_loop, 0, B // 16, unroll=2)
        def _grp(g):
            vecs = [eafv[pl.ds(g * 64 + 16 * m, 16)] for m in range(4)]
            for l in range(16):
                j = g * 16 + l
                vec = vecs[l // 4]
                eb = [_take16(vec, jnp.full((16,), 4 * (l % 4) + k, jnp.int32))
                      for k in range(4)]
                acc = jnp.zeros((16,), _f32)
                for ch in range(8):
                    u = xjv[j, pl.ds(16 * ch, 16)] + xiv[j, pl.ds(16 * ch, 16)]
                    u = u + (eb[0] * wec[0][ch] + eb[1] * wec[1][ch]
                             + eb[2] * wec[2][ch] + eb[3] * wec[3][ch])
                    u = jnp.maximum(u, 0.2 * u)
                    acc = acc + u * attc[ch]
                a16_v[j] = acc
        pltpu.sync_copy(a16_v, a16_hbm.at[pl.ds(base, B)])

    _prefetch(0, bufs[0])

    def _pair(p, _):
        _prefetch(2 * p + 1, bufs[1])
        _compute(2 * p, bufs[0])
        _prefetch(2 * p + 2, bufs[0])
        _compute(2 * p + 1, bufs[1])
        return 0
    lax.fori_loop(0, (E2CH // 32 - 1) // 2, _pair, 0)
    _compute(E2CH // 32 - 1, bufs[0])


@functools.partial(
    pl.kernel,
    out_type=[jax.ShapeDtypeStruct((NC, NP, D), _f32),
              jax.ShapeDtypeStruct((NC, NP, D), _f32)],
    mesh=_mesh,
    compiler_params=_sc_params,
    scratch_types=[
        pltpu.VMEM((B,), jnp.int32),
        pltpu.VMEM((B,), jnp.int32),
        pltpu.VMEM((B,), jnp.int32),
        pltpu.VMEM((B,), jnp.int32),
        pltpu.VMEM((B,), _f32),
        pltpu.VMEM((B,), _f32),
        pltpu.VMEM((B, D), _f32),
        pltpu.VMEM((B, D), _f32),
        pltpu.VMEM_SHARED((NP, D), _f32),
        pltpu.SemaphoreType.DMA,
        pltpu.SemaphoreType.DMA,
    ],
)
def _p2(src_hbm, dst_hbm, ax_hbm, xl_hbm, out_hbm, asum_hbm,
        src0, src1, dst0, dst1, ax0, ax1, xj0, xj1, acc_sh, sj0, sj1):
    cid = lax.axis_index("c")
    sid = lax.axis_index("s")
    w = _wid()
    lane = _lane()

    def _zr(i, _):
        for ch in range(8):
            xj0[i, pl.ds(16 * ch, 16)] = jnp.zeros((16,), _f32)
        return 0
    lax.fori_loop(0, B, _zr, 0)
    for i in range(5):
        pltpu.sync_copy(xj0, acc_sh.at[pl.ds(sid * 640 + i * B, B)])
    plsc.subcore_barrier()

    bufs = ((src0, dst0, ax0, xj0, sj0), (src1, dst1, ax1, xj1, sj1))

    def _prefetch(t, buf):
        srcv, dstv, axv, xjv, sj = buf
        base = (w + 32 * t) * B
        pltpu.sync_copy(src_hbm.at[pl.ds(base, B)], srcv)
        pltpu.sync_copy(dst_hbm.at[pl.ds(base, B)], dstv)
        pltpu.sync_copy(ax_hbm.at[pl.ds(base, B)], axv)
        pltpu.async_copy(xl_hbm.at[srcv], xjv, sj)

    def _compute(t, buf):
        srcv, dstv, axv, xjv, sj = buf
        pltpu.make_async_copy(xl_hbm.at[srcv], xjv, sj).wait()

        @functools.partial(plsc.parallel_loop, 0, B // 16, unroll=2)
        def _grp(g):
            axg = axv[pl.ds(g * 16, 16)]
            for l in range(16):
                j = g * 16 + l
                axb = _take16(axg, jnp.full((16,), l, jnp.int32))
                for ch in range(8):
                    xjv[j, pl.ds(16 * ch, 16)] = xjv[j, pl.ds(16 * ch, 16)] * axb
        pltpu.sync_copy(xjv, acc_sh.at[dstv], add=True)

    _prefetch(0, bufs[0])

    def _pair(p, _):
        _prefetch(2 * p + 1, bufs[1])
        _compute(2 * p, bufs[0])
        _prefetch(2 * p + 2, bufs[0])
        _compute(2 * p + 1, bufs[1])
        return 0
    lax.fori_loop(0, (E2CH // 32 - 1) // 2, _pair, 0)
    _compute(E2CH // 32 - 1, bufs[0])

    plsc.subcore_barrier()
    for i in range(5):
        pltpu.sync_copy(acc_sh.at[pl.ds(sid * 640 + i * B, B)], xj0)
        pltpu.sync_copy(xj0, out_hbm.at[cid, pl.ds(sid * 640 + i * B, B)])
    plsc.subcore_barrier()

    def _zb(i, _):
        for ch in range(8):
            xj0[i, pl.ds(16 * ch, 16)] = jnp.zeros((16,), _f32)
        return 0
    lax.fori_loop(0, B, _zb, 0)
    for i in range(5):
        pltpu.sync_copy(xj0, acc_sh.at[pl.ds(sid * 640 + i * B, B)])
    plsc.subcore_barrier()

    def _chunk_b(t, _):
        base = (w + 32 * t) * B
        pltpu.sync_copy(dst_hbm.at[pl.ds(base, B)], dst0)
        pltpu.sync_copy(ax_hbm.at[pl.ds(base, B)], ax0)

        def _grp(g, _):
            axg = ax0[pl.ds(g * 16, 16)]
            for l in range(16):
                axb = _take16(axg, jnp.full((16,), l, jnp.int32))
                xj0[g * 16 + l, pl.ds(0, 16)] = jnp.where(lane == 0, axb, 0.0)
            return 0
        lax.fori_loop(0, B // 16, _grp, 0)
        pltpu.sync_copy(xj0, acc_sh.at[dst0], add=True)
        return 0
    lax.fori_loop(0, E2CH // 32, _chunk_b, 0)

    plsc.subcore_barrier()
    for i in range(5):
        pltpu.sync_copy(acc_sh.at[pl.ds(sid * 640 + i * B, B)], xj0)
        pltpu.sync_copy(xj0, asum_hbm.at[cid, pl.ds(sid * 640 + i * B, B)])


def _t0_body(h_ref, wl_ref, wr_ref, xl_ref, xr_ref):
    h = h_ref[...]
    dn = (((1,), (0,)), ((), ()))
    xl_ref[...] = lax.dot_general(h, wl_ref[...], dn,
                                  precision=lax.Precision.HIGHEST,
                                  preferred_element_type=_f32)
    xr_ref[...] = lax.dot_general(h, wr_ref[...], dn,
                                  precision=lax.Precision.HIGHEST,
                                  preferred_element_type=_f32)


def _t0(h, wl, wr):
    return pl.pallas_call(
        _t0_body,
        grid=(NP // 256,),
        in_specs=[pl.BlockSpec((256, D), lambda i: (i, 0)),
                  pl.BlockSpec((D, D), lambda i: (0, 0)),
                  pl.BlockSpec((D, D), lambda i: (0, 0))],
        out_specs=[pl.BlockSpec((256, D), lambda i: (i, 0)),
                   pl.BlockSpec((256, D), lambda i: (i, 0))],
        out_shape=[jax.ShapeDtypeStruct((NP, D), _f32),
                   jax.ShapeDtypeStruct((NP, D), _f32)],
    )(h, wl, wr)


def _t1_body(a16_ref, al_ref, gmax_ref, scr):
    i = pl.program_id(0)
    blk = a16_ref[...]
    s = jnp.sum(blk, axis=1)
    al_ref[...] = s

    @pl.when(i == 0)
    def _():
        scr[0, 0] = -3.0e38
    scr[0, 0] = jnp.maximum(scr[0, 0], jnp.max(s))

    @pl.when(i == pl.num_programs(0) - 1)
    def _():
        gmax_ref[0, 0] = scr[0, 0]


def _t1(a16):
    return pl.pallas_call(
        _t1_body,
        grid=(E2P // 4096,),
        in_specs=[pl.BlockSpec((4096, 16), lambda i: (i, 0))],
        out_specs=[pl.BlockSpec((4096,), lambda i: (i,)),
                   pl.BlockSpec(memory_space=pltpu.SMEM)],
        out_shape=[jax.ShapeDtypeStruct((E2P,), _f32),
                   jax.ShapeDtypeStruct((1, 1), _f32)],
        scratch_shapes=[pltpu.SMEM((1, 1), _f32)],
    )(a16)


def _t2_body(al_ref, g_ref, ax_ref):
    ax_ref[...] = jnp.exp(al_ref[...] - g_ref[0, 0])


def _t2(alpha, gmax):
    return pl.pallas_call(
        _t2_body,
        grid=(E2P // 4096,),
        in_specs=[pl.BlockSpec((4096,), lambda i: (i,)),
                  pl.BlockSpec(memory_space=pltpu.SMEM)],
        out_specs=pl.BlockSpec((4096,), lambda i: (i,)),
        out_shape=jax.ShapeDtypeStruct((E2P,), _f32),
    )(alpha, gmax)


def _tla_body(p0_ref, p1_ref, la_ref):
    s = p0_ref[:, :16] + p1_ref[:, :16]
    cnt = jnp.maximum(s[:, 4:5], 1.0)
    col = lax.broadcasted_iota(jnp.int32, s.shape, 1)
    la_ref[...] = jnp.where(col < 4, s / cnt, 0.0)


def _tla(p0, p1):
    return pl.pallas_call(
        _tla_body,
        grid=(NP // 512,),
        in_specs=[pl.BlockSpec((512, D), lambda i: (i, 0)),
                  pl.BlockSpec((512, D), lambda i: (i, 0))],
        out_specs=pl.BlockSpec((512, 16), lambda i: (i, 0)),
        out_shape=jax.ShapeDtypeStruct((NP, 16), _f32),
    )(p0, p1)


def _t3_body(relu, pa_ref, pb_ref, qa_ref, qb_ref, b_ref, h_ref):
    acc = pa_ref[...] + pb_ref[...]
    den = qa_ref[:, 0:1] + qb_ref[:, 0:1] + 1e-16
    h = acc / den + b_ref[...]
    if relu:
        h = jnp.maximum(h, 0.0)
    h_ref[...] = h


def _t3(pa, pb, qa, qb, bias, relu):
    return pl.pallas_call(
        functools.partial(_t3_body, relu),
        grid=(NP // 512,),
        in_specs=[pl.BlockSpec((512, D), lambda i: (i, 0)),
                  pl.BlockSpec((512, D), lambda i: (i, 0)),
                  pl.BlockSpec((512, D), lambda i: (i, 0)),
                  pl.BlockSpec((512, D), lambda i: (i, 0)),
                  pl.BlockSpec((1, D), lambda i: (0, 0))],
        out_specs=pl.BlockSpec((512, D), lambda i: (i, 0)),
        out_shape=jax.ShapeDtypeStruct((NP, D), _f32),
    )(pa, pb, qa, qb, bias)


def _layer(h, src2, dst2, eaf2, wl, wr, we, att, bias, relu):
    xl, xr = _t0(h, wl, wr)
    a16 = _p1(src2, dst2, eaf2, xl, xr, we, att)
    alpha, gmax = _t1(a16)
    aexp = _t2(alpha, gmax)
    part, asw = _p2(src2, dst2, aexp, xl)
    return _t3(part[0], part[1], asw[0], asw[1], bias, relu)


def kernel(x, edge_index, edge_attr,
           Wl1, Wr1, We1, att1, b1,
           Wl2, Wr2, We2, att2, b2,
           Wl3, Wr3, We3, att3, b3):
    src = edge_index[0].astype(jnp.int32)
    dst = edge_index[1].astype(jnp.int32)
    ea = edge_attr.astype(_f32)

    dst0 = jnp.concatenate([dst, jnp.full((E0P - E,), N, jnp.int32)])
    eaf0 = jnp.concatenate([ea.reshape(-1), jnp.zeros(((E0P - E) * 4,), _f32)])
    part0 = _p0(dst0, eaf0)
    la16 = _tla(part0[0], part0[1])
    la = la16[:N, :4]

    loop_idx = jnp.arange(N, dtype=jnp.int32)
    padi = jnp.full((E2P - E2,), N, jnp.int32)
    src2 = jnp.concatenate([src, loop_idx, padi])
    dst2 = jnp.concatenate([dst, loop_idx, padi])
    eaf2 = jnp.concatenate([ea.reshape(-1), la.reshape(-1),
                            jnp.zeros(((E2P - E2) * 4,), _f32)])

    h = jnp.pad(x.astype(_f32), ((0, NP - N), (0, 0)))
    layers = [(Wl1, Wr1, We1, att1, b1, True),
              (Wl2, Wr2, We2, att2, b2, True),
              (Wl3, Wr3, We3, att3, b3, False)]
    for wl, wr, we, att, bias, relu in layers:
        h = _layer(h, src2, dst2, eaf2, wl.astype(_f32), wr.astype(_f32),
                   we.astype(_f32), att.reshape(-1).astype(_f32),
                   bias.reshape(1, D).astype(_f32), relu)
    return h[:N]
